# baseline (speedup 1.0000x reference)
import functools

import jax
import jax.numpy as jnp
import numpy as np
from jax import lax
from jax.experimental import pallas as pl
from jax.experimental.pallas import tpu as pltpu
from jax.experimental.pallas import tpu_sc as plsc

N = 10000
E = 320000
H = 8
D = 16
HD = H * D
FW = HD + 2 * H

NC = 2
NS = 16
NW = NC * NS
EPW = E // NW
CHUNK = 80
NCHUNK = EPW // CHUNK
OUT_RPT = (N // NS) // 8 * 8

_sel = np.zeros((HD, 2 * H), dtype=np.float32)
for _j in range(2 * H):
    _sel[(_j % H) * D:((_j % H) + 1) * D, _j] = 1.0
SEL = _sel

_rex = np.zeros((2 * H, HD), dtype=np.float32)
for _h in range(H):
    _rex[_h, _h * D:(_h + 1) * D] = 1.0
REX = _rex


def _elu(x):
    return jnp.where(x > 0, x, jnp.exp(jnp.minimum(x, 0.0)) - 1.0)


BN = 1000
NB = N // BN

_full = lambda *shape: pl.BlockSpec(shape, lambda i: (0,) * len(shape))


def _dot(a, b):
    return jnp.dot(a, b, preferred_element_type=jnp.float32,
                   precision=lax.Precision.HIGHEST)


def _tc_pre_body(h_ref, w_ref, al_ref, ar_ref, sel_ref, fu_ref, td_ref):
    ft = _dot(h_ref[...], w_ref[...])
    sel = sel_ref[...]
    fu_ref[:, :HD] = ft
    fu_ref[:, HD:] = _dot(ft * al_ref[...], sel)
    td_ref[...] = _dot(ft * ar_ref[...], sel)


def _tc_pre(h, w_fc, al, ar):
    return pl.pallas_call(
        _tc_pre_body,
        grid=(NB,),
        in_specs=[
            pl.BlockSpec((BN, HD), lambda i: (i, 0)),
            _full(HD, HD),
            _full(1, HD),
            _full(1, HD),
            _full(HD, 2 * H),
        ],
        out_specs=[
            pl.BlockSpec((BN, FW), lambda i: (i, 0)),
            pl.BlockSpec((BN, 2 * H), lambda i: (i, 0)),
        ],
        out_shape=[
            jax.ShapeDtypeStruct((N, FW), jnp.float32),
            jax.ShapeDtypeStruct((N, 2 * H), jnp.float32),
        ],
    )(h, w_fc, al, ar, SEL)


def _normalize(cp_ref, rex_ref):
    acc = cp_ref[0, :, :HD] + cp_ref[1, :, :HD]
    s12 = cp_ref[0, :, HD:] + cp_ref[1, :, HD:]
    sbig = _dot(s12, rex_ref[...])
    return _elu(jnp.where(sbig > 0, acc / sbig, 0.0))


def _tc_mid_body(cp_ref, rex_ref, w_ref, al_ref, ar_ref, sel_ref,
                 fu_ref, td_ref):
    h = _normalize(cp_ref, rex_ref)
    ft = _dot(h, w_ref[...])
    sel = sel_ref[...]
    fu_ref[:, :HD] = ft
    fu_ref[:, HD:] = _dot(ft * al_ref[...], sel)
    td_ref[...] = _dot(ft * ar_ref[...], sel)


def _tc_mid(comb, w_fc, al, ar):
    return pl.pallas_call(
        _tc_mid_body,
        grid=(NB,),
        in_specs=[
            pl.BlockSpec((2, BN, FW), lambda i: (0, i, 0)),
            _full(2 * H, HD),
            _full(HD, HD),
            _full(1, HD),
            _full(1, HD),
            _full(HD, 2 * H),
        ],
        out_specs=[
            pl.BlockSpec((BN, FW), lambda i: (i, 0)),
            pl.BlockSpec((BN, 2 * H), lambda i: (i, 0)),
        ],
        out_shape=[
            jax.ShapeDtypeStruct((N, FW), jnp.float32),
            jax.ShapeDtypeStruct((N, 2 * H), jnp.float32),
        ],
    )(comb, REX, w_fc, al, ar, SEL)


def _tc_post_body(cp_ref, rex_ref, w1_ref, b1_ref, w2_ref, b2_ref,
                  w3_ref, b3_ref, out_ref, hg_ref):
    i = pl.program_id(0)
    h = _normalize(cp_ref, rex_ref)
    part = jnp.sum(h, axis=0, keepdims=True)

    @pl.when(i == 0)
    def _init():
        hg_ref[...] = part

    @pl.when(i > 0)
    def _accum():
        hg_ref[...] = hg_ref[...] + part

    @pl.when(i == NB - 1)
    def _mlp():
        hg = hg_ref[...] * (1.0 / N)
        t = jnp.maximum(_dot(hg, w1_ref[...]) + b1_ref[...], 0.0)
        t = jnp.maximum(_dot(t, w2_ref[...]) + b2_ref[...], 0.0)
        out_ref[...] = _dot(t, w3_ref[...]) + b3_ref[...]


def _tc_post(comb, w1, b1, w2, b2, w3, b3):
    return pl.pallas_call(
        _tc_post_body,
        grid=(NB,),
        in_specs=[
            pl.BlockSpec((2, BN, FW), lambda i: (0, i, 0)),
            _full(2 * H, HD),
            _full(HD, 64),
            _full(1, 64),
            _full(64, 64),
            _full(1, 64),
            _full(64, 2),
            _full(1, 2),
        ],
        out_specs=pl.BlockSpec((1, 2), lambda i: (0, 0)),
        out_shape=jax.ShapeDtypeStruct((1, 2), jnp.float32),
        scratch_shapes=[pltpu.VMEM((1, HD), jnp.float32)],
    )(comb, REX, w1, b1, w2, b2, w3, b3)


def _sc_edge_body(src_ref, dst_ref, fu_ref, td_ref,
                  comb_ref,
                  idx_s0, idx_d0, idx_s1, idx_d1,
                  fa0, fa1, a12d0, a12d1,
                  comb_sh, sem0, sem1, isem0, isem1):
    c = lax.axis_index("c")
    s = lax.axis_index("s")
    wid = c * NS + s
    base = wid * EPW

    def load_idx(j, idx_s, idx_d):
        off = base + j * CHUNK
        pltpu.sync_copy(src_ref.at[pl.ds(off, CHUNK)], idx_s)
        pltpu.sync_copy(dst_ref.at[pl.ds(off, CHUNK)], idx_d)

    def load_idx_async(j, idx_s, idx_d, isem):
        off = base + jnp.minimum(j, NCHUNK - 1) * CHUNK
        pltpu.async_copy(src_ref.at[pl.ds(off, CHUNK)], idx_s, isem)
        pltpu.async_copy(dst_ref.at[pl.ds(off, CHUNK)], idx_d, isem)

    def wait_idx(idx_s, idx_d, isem):
        pltpu.make_async_copy(src_ref.at[pl.ds(base, CHUNK)], idx_s,
                              isem).wait()
        pltpu.make_async_copy(dst_ref.at[pl.ds(base, CHUNK)], idx_d,
                              isem).wait()

    def issue(idx_s, idx_d, fa, a12d, sem):
        pltpu.async_copy(fu_ref.at[idx_s], fa, sem)
        pltpu.async_copy(td_ref.at[idx_d], a12d, sem)

    def drain(idx_s, idx_d, fa, a12d, sem):
        pltpu.make_async_copy(fu_ref.at[idx_s], fa, sem).wait()
        pltpu.make_async_copy(td_ref.at[idx_d], a12d, sem).wait()

    def compute_scatter(idx_d, fa, a12d):
        @plsc.parallel_loop(0, CHUNK, step=1, unroll=4)
        def edge_body(e):
            t = fa[e, pl.ds(HD, 2 * H)] + a12d[e, :]
            w = jnp.exp(jnp.maximum(t, t * 0.2))
            fa[e, pl.ds(HD, 2 * H)] = w
            for h in range(H):
                wh = w[h]
                fa[e, pl.ds(h * D, D)] = fa[e, pl.ds(h * D, D)] * wh

        pltpu.sync_copy(fa, comb_sh.at[idx_d], add=True)

    buf0 = (idx_s0, idx_d0, fa0, a12d0, sem0)
    buf1 = (idx_s1, idx_d1, fa1, a12d1, sem1)

    load_idx(0, idx_s0, idx_d0)
    issue(*buf0)
    load_idx_async(1, idx_s1, idx_d1, isem1)

    zvec = jnp.zeros((D,), jnp.float32)

    @plsc.parallel_loop(0, CHUNK, step=1, unroll=4)
    def zero_body(i):
        for j in range(FW // D):
            fa1[i, pl.ds(j * D, D)] = zvec

    r0 = s * OUT_RPT
    nfull = OUT_RPT // CHUNK
    rem = OUT_RPT - nfull * CHUNK
    for j in range(nfull):
        pltpu.sync_copy(fa1, comb_sh.at[pl.ds(r0 + j * CHUNK, CHUNK)])
    pltpu.sync_copy(fa1.at[pl.ds(0, rem)],
                    comb_sh.at[pl.ds(r0 + nfull * CHUNK, rem)])

    @pl.when(s == NS - 1)
    def _zero_tail():
        t0 = NS * OUT_RPT
        pltpu.sync_copy(fa1.at[pl.ds(0, N - t0)], comb_sh.at[pl.ds(t0, N - t0)])

    plsc.subcore_barrier()

    def pair_body(i, _):
        j0 = 2 * i
        drain(*buf0)
        wait_idx(idx_s1, idx_d1, isem1)
        issue(*buf1)
        compute_scatter(idx_d0, fa0, a12d0)
        load_idx_async(j0 + 2, idx_s0, idx_d0, isem0)
        drain(*buf1)
        wait_idx(idx_s0, idx_d0, isem0)
        issue(*buf0)
        compute_scatter(idx_d1, fa1, a12d1)
        load_idx_async(j0 + 3, idx_s1, idx_d1, isem1)
        return 0

    lax.fori_loop(0, (NCHUNK - 1) // 2, pair_body, 0)
    drain(*buf0)
    wait_idx(idx_s1, idx_d1, isem1)
    compute_scatter(idx_d0, fa0, a12d0)
    plsc.subcore_barrier()

    pltpu.sync_copy(comb_sh.at[pl.ds(r0, OUT_RPT)],
                    comb_ref.at[c, pl.ds(r0, OUT_RPT)])

    @pl.when(s == NS - 1)
    def _copy_tail():
        t0 = NS * OUT_RPT
        pltpu.sync_copy(comb_sh.at[pl.ds(t0, N - t0)],
                        comb_ref.at[c, pl.ds(t0, N - t0)])


_sc_edge = pl.kernel(
    _sc_edge_body,
    out_type=jax.ShapeDtypeStruct((NC, N, FW), jnp.float32),
    mesh=plsc.VectorSubcoreMesh(core_axis_name="c", subcore_axis_name="s"),
    compiler_params=pltpu.CompilerParams(use_tc_tiling_on_sc=False),
    scratch_types=[
        pltpu.VMEM((CHUNK,), jnp.int32),
        pltpu.VMEM((CHUNK,), jnp.int32),
        pltpu.VMEM((CHUNK,), jnp.int32),
        pltpu.VMEM((CHUNK,), jnp.int32),
        pltpu.VMEM((CHUNK, FW), jnp.float32),
        pltpu.VMEM((CHUNK, FW), jnp.float32),
        pltpu.VMEM((CHUNK, 2 * H), jnp.float32),
        pltpu.VMEM((CHUNK, 2 * H), jnp.float32),
        pltpu.VMEM_SHARED((N, FW), jnp.float32),
        pltpu.SemaphoreType.DMA,
        pltpu.SemaphoreType.DMA,
        pltpu.SemaphoreType.DMA,
        pltpu.SemaphoreType.DMA,
    ],
)


def kernel(x, edge_index, W_fc1, attn_l1, attn_r1, W_fc2, attn_l2, attn_r2,
           W1, b1, W2, b2, W3, b3):
    src = edge_index[0]
    dst = edge_index[1]
    al1 = attn_l1.reshape(1, HD)
    ar1 = attn_r1.reshape(1, HD)
    al2 = attn_l2.reshape(1, HD)
    ar2 = attn_r2.reshape(1, HD)

    fu1, td1 = _tc_pre(x, W_fc1, al1, ar1)
    comb1 = _sc_edge(src, dst, fu1, td1)
    fu2, td2 = _tc_mid(comb1, W_fc2, al2, ar2)
    comb2 = _sc_edge(src, dst, fu2, td2)
    return _tc_post(comb2, W1, b1.reshape(1, -1), W2, b2.reshape(1, -1),
                    W3, b3.reshape(1, -1))


# speedup vs baseline: 1.0732x; 1.0732x over previous
import functools

import jax
import jax.numpy as jnp
import numpy as np
from jax import lax
from jax.experimental import pallas as pl
from jax.experimental.pallas import tpu as pltpu
from jax.experimental.pallas import tpu_sc as plsc

N = 10000
E = 320000
H = 8
D = 16
HD = H * D
FW = HD + 2 * H

NC = 2
NS = 16
NW = NC * NS
EPW = E // NW
CHUNK = 80
NCHUNK = EPW // CHUNK
OUT_RPT = (N // NS) // 8 * 8

_sel = np.zeros((HD, 2 * H), dtype=np.float32)
for _j in range(2 * H):
    _sel[(_j % H) * D:((_j % H) + 1) * D, _j] = 1.0
SEL = _sel

_rex = np.zeros((2 * H, HD), dtype=np.float32)
for _h in range(H):
    _rex[_h, _h * D:(_h + 1) * D] = 1.0
REX = _rex


def _elu(x):
    return jnp.where(x > 0, x, jnp.exp(jnp.minimum(x, 0.0)) - 1.0)


BN = 1000
NB = N // BN

_full = lambda *shape: pl.BlockSpec(shape, lambda i: (0,) * len(shape))


def _dot(a, b):
    return jnp.dot(a, b, preferred_element_type=jnp.float32,
                   precision=lax.Precision.HIGHEST)


def _tc_pre_body(h_ref, w_ref, al_ref, ar_ref, sel_ref, fu_ref, td_ref):
    ft = _dot(h_ref[...], w_ref[...])
    sel = sel_ref[...]
    fu_ref[:, :HD] = ft
    fu_ref[:, HD:] = _dot(ft * al_ref[...], sel)
    td_ref[...] = _dot(ft * ar_ref[...], sel)


def _tc_pre(h, w_fc, al, ar):
    return pl.pallas_call(
        _tc_pre_body,
        grid=(NB,),
        in_specs=[
            pl.BlockSpec((BN, HD), lambda i: (i, 0)),
            _full(HD, HD),
            _full(1, HD),
            _full(1, HD),
            _full(HD, 2 * H),
        ],
        out_specs=[
            pl.BlockSpec((BN, FW), lambda i: (i, 0)),
            pl.BlockSpec((BN, 2 * H), lambda i: (i, 0)),
        ],
        out_shape=[
            jax.ShapeDtypeStruct((N, FW), jnp.float32),
            jax.ShapeDtypeStruct((N, 2 * H), jnp.float32),
        ],
    )(h, w_fc, al, ar, SEL)


def _normalize(cp_ref, rex_ref):
    acc = cp_ref[0, :, :HD] + cp_ref[1, :, :HD]
    s12 = cp_ref[0, :, HD:] + cp_ref[1, :, HD:]
    sbig = _dot(s12, rex_ref[...])
    return _elu(jnp.where(sbig > 0, acc / sbig, 0.0))


def _tc_mid_body(cp_ref, rex_ref, w_ref, al_ref, ar_ref, sel_ref,
                 fu_ref, td_ref):
    h = _normalize(cp_ref, rex_ref)
    ft = _dot(h, w_ref[...])
    sel = sel_ref[...]
    fu_ref[:, :HD] = ft
    fu_ref[:, HD:] = _dot(ft * al_ref[...], sel)
    td_ref[...] = _dot(ft * ar_ref[...], sel)


def _tc_mid(comb, w_fc, al, ar):
    return pl.pallas_call(
        _tc_mid_body,
        grid=(NB,),
        in_specs=[
            pl.BlockSpec((2, BN, FW), lambda i: (0, i, 0)),
            _full(2 * H, HD),
            _full(HD, HD),
            _full(1, HD),
            _full(1, HD),
            _full(HD, 2 * H),
        ],
        out_specs=[
            pl.BlockSpec((BN, FW), lambda i: (i, 0)),
            pl.BlockSpec((BN, 2 * H), lambda i: (i, 0)),
        ],
        out_shape=[
            jax.ShapeDtypeStruct((N, FW), jnp.float32),
            jax.ShapeDtypeStruct((N, 2 * H), jnp.float32),
        ],
    )(comb, REX, w_fc, al, ar, SEL)


def _tc_post_body(cp_ref, rex_ref, w1_ref, b1_ref, w2_ref, b2_ref,
                  w3_ref, b3_ref, out_ref, hg_ref):
    i = pl.program_id(0)
    h = _normalize(cp_ref, rex_ref)
    part = jnp.sum(h, axis=0, keepdims=True)

    @pl.when(i == 0)
    def _init():
        hg_ref[...] = part

    @pl.when(i > 0)
    def _accum():
        hg_ref[...] = hg_ref[...] + part

    @pl.when(i == NB - 1)
    def _mlp():
        hg = hg_ref[...] * (1.0 / N)
        t = jnp.maximum(_dot(hg, w1_ref[...]) + b1_ref[...], 0.0)
        t = jnp.maximum(_dot(t, w2_ref[...]) + b2_ref[...], 0.0)
        out_ref[...] = _dot(t, w3_ref[...]) + b3_ref[...]


def _tc_post(comb, w1, b1, w2, b2, w3, b3):
    return pl.pallas_call(
        _tc_post_body,
        grid=(NB,),
        in_specs=[
            pl.BlockSpec((2, BN, FW), lambda i: (0, i, 0)),
            _full(2 * H, HD),
            _full(HD, 64),
            _full(1, 64),
            _full(64, 64),
            _full(1, 64),
            _full(64, 2),
            _full(1, 2),
        ],
        out_specs=pl.BlockSpec((1, 2), lambda i: (0, 0)),
        out_shape=jax.ShapeDtypeStruct((1, 2), jnp.float32),
        scratch_shapes=[pltpu.VMEM((1, HD), jnp.float32)],
    )(comb, REX, w1, b1, w2, b2, w3, b3)


def _sc_edge_body(src_ref, dst_ref, fu_ref, td_ref,
                  comb_ref,
                  idx_s0, idx_d0, idx_s1, idx_d1, sidx0, sidx1,
                  fa0, fa1, a12d0, a12d1,
                  comb_sh, sem0, sem1, isem0, isem1, ssem0, ssem1):
    c = lax.axis_index("c")
    s = lax.axis_index("s")
    wid = c * NS + s
    base = wid * EPW

    def load_idx(j, idx_s, idx_d):
        off = base + j * CHUNK
        pltpu.sync_copy(src_ref.at[pl.ds(off, CHUNK)], idx_s)
        pltpu.sync_copy(dst_ref.at[pl.ds(off, CHUNK)], idx_d)

    def load_idx_async(j, idx_s, idx_d, isem):
        off = base + jnp.minimum(j, NCHUNK - 1) * CHUNK
        pltpu.async_copy(src_ref.at[pl.ds(off, CHUNK)], idx_s, isem)
        pltpu.async_copy(dst_ref.at[pl.ds(off, CHUNK)], idx_d, isem)

    def wait_idx(idx_s, idx_d, isem):
        pltpu.make_async_copy(src_ref.at[pl.ds(base, CHUNK)], idx_s,
                              isem).wait()
        pltpu.make_async_copy(dst_ref.at[pl.ds(base, CHUNK)], idx_d,
                              isem).wait()

    def issue(idx_s, idx_d, fa, a12d, sem):
        pltpu.async_copy(fu_ref.at[idx_s], fa, sem)
        pltpu.async_copy(td_ref.at[idx_d], a12d, sem)

    def drain(idx_s, idx_d, fa, a12d, sem):
        pltpu.make_async_copy(fu_ref.at[idx_s], fa, sem).wait()
        pltpu.make_async_copy(td_ref.at[idx_d], a12d, sem).wait()

    def compute(fa, a12d):
        @plsc.parallel_loop(0, CHUNK, step=1, unroll=4)
        def edge_body(e):
            t = fa[e, pl.ds(HD, 2 * H)] + a12d[e, :]
            w = jnp.exp(jnp.maximum(t, t * 0.2))
            fa[e, pl.ds(HD, 2 * H)] = w
            for h in range(H):
                wh = w[h]
                fa[e, pl.ds(h * D, D)] = fa[e, pl.ds(h * D, D)] * wh

    def scatter_async(idx_d, sidx, fa, ssem):
        for k in range(CHUNK // 16):
            sidx[pl.ds(k * 16, 16)] = idx_d[pl.ds(k * 16, 16)]
        pltpu.async_copy(fa, comb_sh.at[sidx], ssem, add=True)

    def wait_scatter(sidx, fa, ssem):
        pltpu.make_async_copy(fa, comb_sh.at[sidx], ssem).wait()

    buf0 = (idx_s0, idx_d0, fa0, a12d0, sem0)
    buf1 = (idx_s1, idx_d1, fa1, a12d1, sem1)

    load_idx(0, idx_s0, idx_d0)
    issue(*buf0)
    load_idx_async(1, idx_s1, idx_d1, isem1)

    zvec = jnp.zeros((D,), jnp.float32)

    @plsc.parallel_loop(0, CHUNK, step=1, unroll=4)
    def zero_body(i):
        for j in range(FW // D):
            fa1[i, pl.ds(j * D, D)] = zvec

    r0 = s * OUT_RPT
    nfull = OUT_RPT // CHUNK
    rem = OUT_RPT - nfull * CHUNK
    for j in range(nfull):
        pltpu.sync_copy(fa1, comb_sh.at[pl.ds(r0 + j * CHUNK, CHUNK)])
    pltpu.sync_copy(fa1.at[pl.ds(0, rem)],
                    comb_sh.at[pl.ds(r0 + nfull * CHUNK, rem)])

    @pl.when(s == NS - 1)
    def _zero_tail():
        t0 = NS * OUT_RPT
        pltpu.sync_copy(fa1.at[pl.ds(0, N - t0)], comb_sh.at[pl.ds(t0, N - t0)])

    plsc.subcore_barrier()

    for k in range(CHUNK // 16):
        sidx1[pl.ds(k * 16, 16)] = idx_d0[pl.ds(k * 16, 16)]
    pltpu.async_copy(fa1, comb_sh.at[sidx1], ssem1, add=True)

    def pair_body(i, _):
        j0 = 2 * i
        drain(*buf0)
        wait_idx(idx_s1, idx_d1, isem1)
        wait_scatter(sidx1, fa1, ssem1)
        issue(*buf1)
        compute(fa0, a12d0)
        scatter_async(idx_d0, sidx0, fa0, ssem0)
        load_idx_async(j0 + 2, idx_s0, idx_d0, isem0)
        drain(*buf1)
        wait_idx(idx_s0, idx_d0, isem0)
        wait_scatter(sidx0, fa0, ssem0)
        issue(*buf0)
        compute(fa1, a12d1)
        scatter_async(idx_d1, sidx1, fa1, ssem1)
        load_idx_async(j0 + 3, idx_s1, idx_d1, isem1)
        return 0

    lax.fori_loop(0, (NCHUNK - 1) // 2, pair_body, 0)
    drain(*buf0)
    wait_idx(idx_s1, idx_d1, isem1)
    wait_scatter(sidx1, fa1, ssem1)
    compute(fa0, a12d0)
    pltpu.sync_copy(fa0, comb_sh.at[idx_d0], add=True)
    plsc.subcore_barrier()

    pltpu.sync_copy(comb_sh.at[pl.ds(r0, OUT_RPT)],
                    comb_ref.at[c, pl.ds(r0, OUT_RPT)])

    @pl.when(s == NS - 1)
    def _copy_tail():
        t0 = NS * OUT_RPT
        pltpu.sync_copy(comb_sh.at[pl.ds(t0, N - t0)],
                        comb_ref.at[c, pl.ds(t0, N - t0)])


_sc_edge = pl.kernel(
    _sc_edge_body,
    out_type=jax.ShapeDtypeStruct((NC, N, FW), jnp.float32),
    mesh=plsc.VectorSubcoreMesh(core_axis_name="c", subcore_axis_name="s"),
    compiler_params=pltpu.CompilerParams(use_tc_tiling_on_sc=False),
    scratch_types=[
        pltpu.VMEM((CHUNK,), jnp.int32),
        pltpu.VMEM((CHUNK,), jnp.int32),
        pltpu.VMEM((CHUNK,), jnp.int32),
        pltpu.VMEM((CHUNK,), jnp.int32),
        pltpu.VMEM((CHUNK,), jnp.int32),
        pltpu.VMEM((CHUNK,), jnp.int32),
        pltpu.VMEM((CHUNK, FW), jnp.float32),
        pltpu.VMEM((CHUNK, FW), jnp.float32),
        pltpu.VMEM((CHUNK, 2 * H), jnp.float32),
        pltpu.VMEM((CHUNK, 2 * H), jnp.float32),
        pltpu.VMEM_SHARED((N, FW), jnp.float32),
        pltpu.SemaphoreType.DMA,
        pltpu.SemaphoreType.DMA,
        pltpu.SemaphoreType.DMA,
        pltpu.SemaphoreType.DMA,
        pltpu.SemaphoreType.DMA,
        pltpu.SemaphoreType.DMA,
    ],
)


def kernel(x, edge_index, W_fc1, attn_l1, attn_r1, W_fc2, attn_l2, attn_r2,
           W1, b1, W2, b2, W3, b3):
    src = edge_index[0]
    dst = edge_index[1]
    al1 = attn_l1.reshape(1, HD)
    ar1 = attn_r1.reshape(1, HD)
    al2 = attn_l2.reshape(1, HD)
    ar2 = attn_r2.reshape(1, HD)

    fu1, td1 = _tc_pre(x, W_fc1, al1, ar1)
    comb1 = _sc_edge(src, dst, fu1, td1)
    fu2, td2 = _tc_mid(comb1, W_fc2, al2, ar2)
    comb2 = _sc_edge(src, dst, fu2, td2)
    return _tc_post(comb2, W1, b1.reshape(1, -1), W2, b2.reshape(1, -1),
                    W3, b3.reshape(1, -1))
